# baseline (speedup 1.0000x reference)
import functools

import jax
import jax.numpy as jnp
from jax import lax
from jax.experimental import pallas as pl
from jax.experimental.pallas import tpu as pltpu
from jax.experimental.pallas import tpu_sc as plsc

_N = 8192
_K = 8192
_D = 256
_TM = 1024
_TK = 2048
_NT = _N // _TM
_KT = _K // _TK
_BETA = 0.25


def _dist_argmin_body(zt_ref, e_ref, idx_ref, loss_ref,
                      run_s, pick_s, amin_s, esq_s, zt2_s, zsq_s):
    t = pl.program_id(0)
    k = pl.program_id(1)

    e_blk = e_ref[...]

    @pl.when(t == 0)
    def _strip_esq():
        esq_s[pl.ds(k * _TK, _TK), :] = jnp.sum(e_blk * e_blk, axis=1,
                                                keepdims=True)

    @pl.when(k == 0)
    def _tile_z():
        zt_blk = zt_ref[0]
        zt2_s[...] = zt_blk + zt_blk
        zsq_s[...] = jnp.sum(zt_blk * zt_blk, axis=0, keepdims=True)

    scores2 = jnp.dot(e_blk, zt2_s[...])
    esq = esq_s[pl.ds(k * _TK, _TK), :]
    dist = (zsq_s[...] + esq) - scores2

    m = jnp.min(dist, axis=0, keepdims=True)
    ids = lax.broadcasted_iota(jnp.int32, (_TK, _TM), 0)
    am = jnp.min(jnp.where(dist == m, ids, jnp.int32(2**30)),
                 axis=0, keepdims=True) + k * _TK
    mb = m.astype(jnp.bfloat16).astype(jnp.float32)

    @pl.when(k == 0)
    def _init():
        run_s[...] = mb
        amin_s[...] = am
        pick_s[...] = m

    @pl.when(k > 0)
    def _update():
        r = run_s[...]
        lt = m < r
        take = lt | ((m == r) & (am < amin_s[...]))
        amin_s[...] = jnp.where(take, am, amin_s[...])
        pick_s[...] = jnp.where(take, m, pick_s[...])
        run_s[...] = jnp.where(lt, mb, r)

    @pl.when((t == 0) & (k == 0))
    def _zero_loss():
        loss_ref[0, 0] = 0.0

    @pl.when(k == _KT - 1)
    def _final():
        idx_ref[0, 0, :] = amin_s[0, :]
        loss_ref[0, 0] += jnp.sum(pick_s[...])

    @pl.when((t == _NT - 1) & (k == _KT - 1))
    def _finish_loss():
        loss_ref[0, 0] = loss_ref[0, 0] * ((1.0 + _BETA) / (_N * _D))


def _dist_argmin(zt3, e):
    idx3, loss = pl.pallas_call(
        _dist_argmin_body,
        grid=(_NT, _KT),
        in_specs=[
            pl.BlockSpec((1, _D, _TM), lambda t, k: (t, 0, 0)),
            pl.BlockSpec((_TK, _D), lambda t, k: (k, 0)),
        ],
        out_specs=[
            pl.BlockSpec((1, 1, _TM), lambda t, k: (t, 0, 0)),
            pl.BlockSpec((1, 1), lambda t, k: (0, 0),
                         memory_space=pltpu.SMEM),
        ],
        out_shape=[
            jax.ShapeDtypeStruct((_NT, 1, _TM), jnp.int32),
            jax.ShapeDtypeStruct((1, 1), jnp.float32),
        ],
        scratch_shapes=[
            pltpu.VMEM((1, _TM), jnp.float32),
            pltpu.VMEM((1, _TM), jnp.float32),
            pltpu.VMEM((1, _TM), jnp.int32),
            pltpu.VMEM((_K, 1), jnp.float32),
            pltpu.VMEM((_D, _TM), jnp.float32),
            pltpu.VMEM((1, _TM), jnp.float32),
        ],
    )(zt3, e)
    return idx3.reshape(_N), loss.reshape(())


_NC = 2
_NS = 16
_NW = _NC * _NS
_BPW = _N // _NW


def _sc_gather_body(table_hbm, idx_hbm, out_hbm, idx_v, rows_v, sem):
    wid = lax.axis_index("s") * _NC + lax.axis_index("c")
    base = wid * _BPW
    pltpu.sync_copy(idx_hbm.at[pl.ds(base, _BPW)], idx_v)
    pltpu.async_copy(table_hbm.at[idx_v], rows_v, sem).wait()
    pltpu.sync_copy(rows_v, out_hbm.at[pl.ds(base, _BPW)])


@functools.cache
def _sc_gather():
    return pl.kernel(
        _sc_gather_body,
        mesh=plsc.VectorSubcoreMesh(core_axis_name="c", subcore_axis_name="s"),
        out_type=jax.ShapeDtypeStruct((_N, _D), jnp.float32),
        scratch_types=[
            pltpu.VMEM((_BPW,), jnp.int32),
            pltpu.VMEM((_BPW, _D), jnp.float32),
            pltpu.SemaphoreType.DMA,
        ],
    )


def kernel(x, embed_weight):
    zt3 = x.reshape(_NT, _D, _TM)
    idx, loss = _dist_argmin(zt3, embed_weight)
    zq = _sc_gather()(embed_weight, idx)
    zq = jnp.transpose(zq.reshape(8, 32, 32, _D), (0, 3, 1, 2))
    return (zq, loss)


# speedup vs baseline: 1.0239x; 1.0239x over previous
import functools

import jax
import jax.numpy as jnp
from jax import lax
from jax.experimental import pallas as pl
from jax.experimental.pallas import tpu as pltpu
from jax.experimental.pallas import tpu_sc as plsc

_N = 8192
_K = 8192
_D = 256
_TM = 1024
_TK = 2048
_NT = _N // _TM
_KT = _K // _TK
_BETA = 0.25


def _dist_argmin_body(zt_ref, e_ref, idx_ref, loss_ref,
                      run_s, pick_s, amin_s, esq_s):
    t = pl.program_id(0)
    k = pl.program_id(1)

    e_blk = e_ref[...]

    @pl.when(t == 0)
    def _strip_esq():
        esq_s[pl.ds(k * _TK, _TK), :] = jnp.sum(e_blk * e_blk, axis=1,
                                                keepdims=True)

    zt_blk = zt_ref[0]
    scores2 = jnp.dot(e_blk, zt_blk + zt_blk)
    zsq = jnp.sum(zt_blk * zt_blk, axis=0, keepdims=True)
    esq = esq_s[pl.ds(k * _TK, _TK), :]
    dist = (zsq + esq) - scores2

    m = jnp.min(dist, axis=0, keepdims=True)
    ids = lax.broadcasted_iota(jnp.int32, (_TK, _TM), 0)
    am = jnp.min(jnp.where(dist == m, ids, jnp.int32(2**30)),
                 axis=0, keepdims=True) + k * _TK
    mb = m.astype(jnp.bfloat16).astype(jnp.float32)

    @pl.when(k == 0)
    def _init():
        run_s[...] = mb
        amin_s[...] = am
        pick_s[...] = m

    @pl.when(k > 0)
    def _update():
        r = run_s[...]
        lt = m < r
        take = lt | ((m == r) & (am < amin_s[...]))
        amin_s[...] = jnp.where(take, am, amin_s[...])
        pick_s[...] = jnp.where(take, m, pick_s[...])
        run_s[...] = jnp.where(lt, mb, r)

    @pl.when((t == 0) & (k == 0))
    def _zero_loss():
        loss_ref[0, 0] = 0.0

    @pl.when(k == _KT - 1)
    def _final():
        idx_ref[0, 0, :] = amin_s[0, :]
        loss_ref[0, 0] += jnp.sum(pick_s[...])

    @pl.when((t == _NT - 1) & (k == _KT - 1))
    def _finish_loss():
        loss_ref[0, 0] = loss_ref[0, 0] * ((1.0 + _BETA) / (_N * _D))


def _dist_argmin(zt3, e):
    idx3, loss = pl.pallas_call(
        _dist_argmin_body,
        grid=(_NT, _KT),
        in_specs=[
            pl.BlockSpec((1, _D, _TM), lambda t, k: (t, 0, 0)),
            pl.BlockSpec((_TK, _D), lambda t, k: (k, 0)),
        ],
        out_specs=[
            pl.BlockSpec((1, 1, _TM), lambda t, k: (t, 0, 0)),
            pl.BlockSpec((1, 1), lambda t, k: (0, 0),
                         memory_space=pltpu.SMEM),
        ],
        out_shape=[
            jax.ShapeDtypeStruct((_NT, 1, _TM), jnp.int32),
            jax.ShapeDtypeStruct((1, 1), jnp.float32),
        ],
        scratch_shapes=[
            pltpu.VMEM((1, _TM), jnp.float32),
            pltpu.VMEM((1, _TM), jnp.float32),
            pltpu.VMEM((1, _TM), jnp.int32),
            pltpu.VMEM((_K, 1), jnp.float32),
        ],
    )(zt3, e)
    return idx3.reshape(_N), loss.reshape(())


_NC = 2
_NS = 16
_NW = _NC * _NS
_BPW = _N // _NW


def _sc_gather_body(table_hbm, idx_hbm, out_hbm, idx_v, rows_v, sem):
    wid = lax.axis_index("s") * _NC + lax.axis_index("c")
    base = wid * _BPW
    pltpu.sync_copy(idx_hbm.at[pl.ds(base, _BPW)], idx_v)
    pltpu.async_copy(table_hbm.at[idx_v], rows_v, sem).wait()
    pltpu.sync_copy(rows_v, out_hbm.at[pl.ds(base, _BPW)])


@functools.cache
def _sc_gather():
    return pl.kernel(
        _sc_gather_body,
        mesh=plsc.VectorSubcoreMesh(core_axis_name="c", subcore_axis_name="s"),
        out_type=jax.ShapeDtypeStruct((_N, _D), jnp.float32),
        scratch_types=[
            pltpu.VMEM((_BPW,), jnp.int32),
            pltpu.VMEM((_BPW, _D), jnp.float32),
            pltpu.SemaphoreType.DMA,
        ],
    )


def kernel(x, embed_weight):
    zt3 = x.reshape(_NT, _D, _TM)
    idx, loss = _dist_argmin(zt3, embed_weight)
    zq = _sc_gather()(embed_weight, idx)
    zq = jnp.transpose(zq.reshape(8, 32, 32, _D), (0, 3, 1, 2))
    return (zq, loss)
